# baseline (speedup 1.0000x reference)
import jax
import jax.numpy as jnp
from jax.experimental import pallas as pl
from jax.experimental.pallas import tpu as pltpu

B = 1024
IN = 64
H = 512
MAX_PONDER = 20
EPS = 0.01

_DN_TT = (((1,), (1,)), ((), ()))
_DN_NN = (((1,), (0,)), ((), ()))


def _act_body(x_ref, wih_ref, bih_ref, whh_hbm_ref, bhh_ref, wp_ref, bp_ref,
              wfc_ref, bfc_ref, out_ref, pc_ref,
              base_ref, hx_ref, ahx_ref, ah_ref, spc_ref, sc_ref,
              whh_ref, whh_sem):
    f32 = jnp.float32
    whh_cp = pltpu.make_async_copy(whh_hbm_ref, whh_ref, whh_sem)
    whh_cp.start()
    bp = bp_ref[0]
    bias_col = jnp.reshape(bih_ref[:] + bhh_ref[:], (1, H)).T
    wflag_col = wih_ref[:, IN:IN + 1]
    base_ref[:] = jax.lax.dot_general(
        wih_ref[:, :IN], x_ref[:], _DN_TT, preferred_element_type=f32
    ) + bias_col

    hx0 = jnp.tanh(base_ref[:])
    hx_ref[:] = hx0
    h0 = jax.nn.sigmoid(jax.lax.dot_general(
        wp_ref[:], hx0, _DN_NN, preferred_element_type=f32) + bp)
    p0 = h0 - jnp.maximum(h0 - 1.0, 0.0)
    ahx_ref[:] = (1.0 + p0) * hx0
    ah_ref[:] = h0
    spc_ref[:] = jnp.zeros((1, B), f32)
    sc_ref[:] = jnp.ones((1, B), f32)
    done0 = (jnp.min(h0) >= (1.0 - EPS)).astype(jnp.int32)
    whh_cp.wait()

    def cond(carry):
        i, done = carry
        return jnp.logical_and(i < MAX_PONDER, done == 0)

    def step(carry):
        i, _ = carry
        accum_h = ah_ref[:]
        sel = accum_h < (1.0 - EPS)
        spc_ref[:] = jnp.where(sel, accum_h, spc_ref[:])
        hx = jnp.tanh(
            base_ref[:]
            + i.astype(f32) * wflag_col
            + jax.lax.dot_general(whh_ref[:], hx_ref[:], _DN_NN,
                                  preferred_element_type=f32)
        )
        hx_ref[:] = hx
        h = jax.nn.sigmoid(jax.lax.dot_general(
            wp_ref[:], hx, _DN_NN, preferred_element_type=f32) + bp)
        accum_h_new = accum_h + jnp.where(sel, h, 0.0)
        p = h - jnp.maximum(accum_h_new - 1.0, 0.0)
        coef = jnp.where(sel, 1.0 + p, 0.0)
        ahx_ref[:] = ahx_ref[:] + coef * hx
        ah_ref[:] = accum_h_new
        sc_ref[:] = sc_ref[:] + jnp.where(sel, 1.0, 0.0)
        all_halted = jnp.min(accum_h_new) >= (1.0 - EPS)
        return i + 1, all_halted.astype(jnp.int32)

    jax.lax.while_loop(cond, step, (1, done0))

    num = jax.lax.dot_general(
        wfc_ref[:], ahx_ref[:], _DN_NN, preferred_element_type=f32)
    out_ref[:] = jnp.reshape(num / sc_ref[:] + bfc_ref[0], (B,))
    pc_ref[:] = jnp.reshape(-spc_ref[:], (B,))


@jax.jit
def _act_kernel(x, W_ih, b_ih, W_hh, b_hh, W_p, b_p, W_fc, b_fc):
    return pl.pallas_call(
        _act_body,
        out_shape=(
            jax.ShapeDtypeStruct((B,), jnp.float32),
            jax.ShapeDtypeStruct((B,), jnp.float32),
        ),
        in_specs=[
            pl.BlockSpec(memory_space=pltpu.VMEM),
            pl.BlockSpec(memory_space=pltpu.VMEM),
            pl.BlockSpec(memory_space=pltpu.VMEM),
            pl.BlockSpec(memory_space=pl.ANY),
            pl.BlockSpec(memory_space=pltpu.VMEM),
            pl.BlockSpec(memory_space=pltpu.VMEM),
            pl.BlockSpec(memory_space=pltpu.SMEM),
            pl.BlockSpec(memory_space=pltpu.VMEM),
            pl.BlockSpec(memory_space=pltpu.SMEM),
        ],
        scratch_shapes=[
            pltpu.VMEM((H, B), jnp.float32),
            pltpu.VMEM((H, B), jnp.float32),
            pltpu.VMEM((H, B), jnp.float32),
            pltpu.VMEM((1, B), jnp.float32),
            pltpu.VMEM((1, B), jnp.float32),
            pltpu.VMEM((1, B), jnp.float32),
            pltpu.VMEM((H, H), jnp.float32),
            pltpu.SemaphoreType.DMA,
        ],
    )(x, W_ih, b_ih, W_hh, b_hh, W_p, b_p, W_fc, b_fc)


def kernel(x, W_ih, b_ih, W_hh, b_hh, W_p, b_p, W_fc, b_fc):
    return _act_kernel(x, W_ih, b_ih, W_hh, b_hh, W_p, b_p, W_fc, b_fc)


# speedup vs baseline: 1.0454x; 1.0454x over previous
import jax
import jax.numpy as jnp
from jax.experimental import pallas as pl
from jax.experimental.pallas import tpu as pltpu

B = 1024
IN = 64
H = 512
MAX_PONDER = 20
EPS = 0.01

_DN_TT = (((1,), (1,)), ((), ()))
_DN_NN = (((1,), (0,)), ((), ()))


def _act_body(x_hbm, wih_hbm, bih_hbm, whh_hbm, bhh_hbm, wp_hbm, bp_ref,
              wfc_hbm, bfc_ref, out_ref, pc_ref,
              base_ref, hx_ref, ahx_ref, ah_ref, spc_ref, sc_ref,
              x_ref, wih_ref, bih_ref, whh_ref, bhh_ref, wp_ref, wfc_ref,
              sem_x, sem_wih, sem_bih, sem_whh, sem_bhh, sem_wp, sem_wfc):
    f32 = jnp.float32
    cp_x = pltpu.make_async_copy(x_hbm, x_ref, sem_x)
    cp_wih = pltpu.make_async_copy(wih_hbm, wih_ref, sem_wih)
    cp_bih = pltpu.make_async_copy(bih_hbm, bih_ref, sem_bih)
    cp_bhh = pltpu.make_async_copy(bhh_hbm, bhh_ref, sem_bhh)
    cp_wp = pltpu.make_async_copy(wp_hbm, wp_ref, sem_wp)
    cp_wfc = pltpu.make_async_copy(wfc_hbm, wfc_ref, sem_wfc)
    cp_whh = pltpu.make_async_copy(whh_hbm, whh_ref, sem_whh)
    cp_x.start()
    cp_wih.start()
    cp_bih.start()
    cp_bhh.start()
    cp_wp.start()
    cp_wfc.start()
    cp_whh.start()

    bp = bp_ref[0]
    cp_bih.wait()
    cp_bhh.wait()
    cp_x.wait()
    cp_wih.wait()
    bias_col = jnp.reshape(bih_ref[:] + bhh_ref[:], (1, H)).T
    wflag_col = wih_ref[:, IN:IN + 1]
    base_ref[:] = jax.lax.dot_general(
        wih_ref[:, :IN], x_ref[:], _DN_TT, preferred_element_type=f32
    ) + bias_col

    hx0 = jnp.tanh(base_ref[:])
    hx_ref[:] = hx0
    cp_wp.wait()
    h0 = jax.nn.sigmoid(jax.lax.dot_general(
        wp_ref[:], hx0, _DN_NN, preferred_element_type=f32) + bp)
    p0 = h0 - jnp.maximum(h0 - 1.0, 0.0)
    ahx_ref[:] = (1.0 + p0) * hx0
    ah_ref[:] = h0
    spc_ref[:] = jnp.zeros((1, B), f32)
    sc_ref[:] = jnp.ones((1, B), f32)
    done0 = (jnp.min(h0) >= (1.0 - EPS)).astype(jnp.int32)
    cp_whh.wait()

    def cond(carry):
        i, done = carry
        return jnp.logical_and(i < MAX_PONDER, done == 0)

    def step(carry):
        i, _ = carry
        accum_h = ah_ref[:]
        sel = accum_h < (1.0 - EPS)
        spc_ref[:] = jnp.where(sel, accum_h, spc_ref[:])
        hx = jnp.tanh(
            base_ref[:]
            + i.astype(f32) * wflag_col
            + jax.lax.dot_general(whh_ref[:], hx_ref[:], _DN_NN,
                                  preferred_element_type=f32)
        )
        hx_ref[:] = hx
        h = jax.nn.sigmoid(jax.lax.dot_general(
            wp_ref[:], hx, _DN_NN, preferred_element_type=f32) + bp)
        accum_h_new = accum_h + jnp.where(sel, h, 0.0)
        p = h - jnp.maximum(accum_h_new - 1.0, 0.0)
        coef = jnp.where(sel, 1.0 + p, 0.0)
        ahx_ref[:] = ahx_ref[:] + coef * hx
        ah_ref[:] = accum_h_new
        sc_ref[:] = sc_ref[:] + jnp.where(sel, 1.0, 0.0)
        all_halted = jnp.min(accum_h_new) >= (1.0 - EPS)
        return i + 1, all_halted.astype(jnp.int32)

    jax.lax.while_loop(cond, step, (1, done0))

    cp_wfc.wait()
    num = jax.lax.dot_general(
        wfc_ref[:], ahx_ref[:], _DN_NN, preferred_element_type=f32)
    out_ref[:] = jnp.reshape(num / sc_ref[:] + bfc_ref[0], (B,))
    pc_ref[:] = jnp.reshape(-spc_ref[:], (B,))


@jax.jit
def _act_kernel(x, W_ih, b_ih, W_hh, b_hh, W_p, b_p, W_fc, b_fc):
    return pl.pallas_call(
        _act_body,
        out_shape=(
            jax.ShapeDtypeStruct((B,), jnp.float32),
            jax.ShapeDtypeStruct((B,), jnp.float32),
        ),
        in_specs=[
            pl.BlockSpec(memory_space=pl.ANY),
            pl.BlockSpec(memory_space=pl.ANY),
            pl.BlockSpec(memory_space=pl.ANY),
            pl.BlockSpec(memory_space=pl.ANY),
            pl.BlockSpec(memory_space=pl.ANY),
            pl.BlockSpec(memory_space=pl.ANY),
            pl.BlockSpec(memory_space=pltpu.SMEM),
            pl.BlockSpec(memory_space=pl.ANY),
            pl.BlockSpec(memory_space=pltpu.SMEM),
        ],
        scratch_shapes=[
            pltpu.VMEM((H, B), jnp.float32),
            pltpu.VMEM((H, B), jnp.float32),
            pltpu.VMEM((H, B), jnp.float32),
            pltpu.VMEM((1, B), jnp.float32),
            pltpu.VMEM((1, B), jnp.float32),
            pltpu.VMEM((1, B), jnp.float32),
            pltpu.VMEM((B, IN), jnp.float32),
            pltpu.VMEM((H, IN + 1), jnp.float32),
            pltpu.VMEM((H,), jnp.float32),
            pltpu.VMEM((H, H), jnp.float32),
            pltpu.VMEM((H,), jnp.float32),
            pltpu.VMEM((1, H), jnp.float32),
            pltpu.VMEM((1, H), jnp.float32),
            pltpu.SemaphoreType.DMA,
            pltpu.SemaphoreType.DMA,
            pltpu.SemaphoreType.DMA,
            pltpu.SemaphoreType.DMA,
            pltpu.SemaphoreType.DMA,
            pltpu.SemaphoreType.DMA,
            pltpu.SemaphoreType.DMA,
        ],
    )(x, W_ih, b_ih, W_hh, b_hh, W_p, b_p, W_fc, b_fc)


def kernel(x, W_ih, b_ih, W_hh, b_hh, W_p, b_p, W_fc, b_fc):
    return _act_kernel(x, W_ih, b_ih, W_hh, b_hh, W_p, b_p, W_fc, b_fc)


# speedup vs baseline: 1.0463x; 1.0008x over previous
import jax
import jax.numpy as jnp
from jax.experimental import pallas as pl
from jax.experimental.pallas import tpu as pltpu

B = 1024
IN = 64
H = 512
MAX_PONDER = 20
EPS = 0.01

_DN_TT = (((1,), (1,)), ((), ()))
_DN_NN = (((1,), (0,)), ((), ()))


def _act_body(x_ref, wih_ref, bih_ref, whh_ref, bhh_ref, wp_ref, bp_ref,
              wfc_ref, bfc_ref, out_ref, pc_ref,
              base_ref, hx_ref, ahx_ref, ah_ref, spc_ref, sc_ref):
    f32 = jnp.float32
    bp = bp_ref[0]
    bias_col = jnp.reshape(bih_ref[:] + bhh_ref[:], (1, H)).T
    wflag_col = wih_ref[:, IN:IN + 1]
    base_ref[:] = jax.lax.dot_general(
        wih_ref[:, :IN], x_ref[:], _DN_TT, preferred_element_type=f32
    ) + bias_col

    hx0 = jnp.tanh(base_ref[:])
    hx_ref[:] = hx0
    h0 = jax.nn.sigmoid(jax.lax.dot_general(
        wp_ref[:], hx0, _DN_NN, preferred_element_type=f32) + bp)
    p0 = h0 - jnp.maximum(h0 - 1.0, 0.0)
    ahx_ref[:] = (1.0 + p0) * hx0
    ah_ref[:] = h0
    spc_ref[:] = jnp.zeros((1, B), f32)
    sc_ref[:] = jnp.ones((1, B), f32)
    done0 = (jnp.min(h0) >= (1.0 - EPS)).astype(jnp.int32)

    def cond(carry):
        i, done = carry
        return jnp.logical_and(i < MAX_PONDER, done == 0)

    def step(carry):
        i, _ = carry
        accum_h = ah_ref[:]
        sel = accum_h < (1.0 - EPS)
        spc_ref[:] = jnp.where(sel, accum_h, spc_ref[:])
        hx = jnp.tanh(
            base_ref[:]
            + i.astype(f32) * wflag_col
            + jax.lax.dot_general(whh_ref[:], hx_ref[:], _DN_NN,
                                  preferred_element_type=f32)
        )
        hx_ref[:] = hx
        h = jax.nn.sigmoid(jax.lax.dot_general(
            wp_ref[:], hx, _DN_NN, preferred_element_type=f32) + bp)
        accum_h_new = accum_h + jnp.where(sel, h, 0.0)
        p = h - jnp.maximum(accum_h_new - 1.0, 0.0)
        coef = jnp.where(sel, 1.0 + p, 0.0)
        ahx_ref[:] = ahx_ref[:] + coef * hx
        ah_ref[:] = accum_h_new
        sc_ref[:] = sc_ref[:] + jnp.where(sel, 1.0, 0.0)
        all_halted = jnp.min(accum_h_new) >= (1.0 - EPS)
        return i + 1, all_halted.astype(jnp.int32)

    jax.lax.while_loop(cond, step, (1, done0))

    num = jax.lax.dot_general(
        wfc_ref[:], ahx_ref[:], _DN_NN, preferred_element_type=f32)
    out_ref[:] = jnp.reshape(num / sc_ref[:] + bfc_ref[0], (B,))
    pc_ref[:] = jnp.reshape(-spc_ref[:], (B,))


@jax.jit
def _act_kernel(x, W_ih, b_ih, W_hh, b_hh, W_p, b_p, W_fc, b_fc):
    return pl.pallas_call(
        _act_body,
        out_shape=(
            jax.ShapeDtypeStruct((B,), jnp.float32),
            jax.ShapeDtypeStruct((B,), jnp.float32),
        ),
        in_specs=[
            pl.BlockSpec(memory_space=pltpu.VMEM),
            pl.BlockSpec(memory_space=pltpu.VMEM),
            pl.BlockSpec(memory_space=pltpu.VMEM),
            pl.BlockSpec(memory_space=pltpu.VMEM),
            pl.BlockSpec(memory_space=pltpu.VMEM),
            pl.BlockSpec(memory_space=pltpu.VMEM),
            pl.BlockSpec(memory_space=pltpu.SMEM),
            pl.BlockSpec(memory_space=pltpu.VMEM),
            pl.BlockSpec(memory_space=pltpu.SMEM),
        ],
        scratch_shapes=[
            pltpu.VMEM((H, B), jnp.float32),
            pltpu.VMEM((H, B), jnp.float32),
            pltpu.VMEM((H, B), jnp.float32),
            pltpu.VMEM((1, B), jnp.float32),
            pltpu.VMEM((1, B), jnp.float32),
            pltpu.VMEM((1, B), jnp.float32),
        ],
    )(x, W_ih, b_ih, W_hh, b_hh, W_p, b_p, W_fc, b_fc)


def kernel(x, W_ih, b_ih, W_hh, b_hh, W_p, b_p, W_fc, b_fc):
    return _act_kernel(x, W_ih, b_ih, W_hh, b_hh, W_p, b_p, W_fc, b_fc)
